# baseline (speedup 1.0000x reference)
import functools

import jax
import jax.numpy as jnp
from jax import lax
from jax.experimental import pallas as pl
from jax.experimental.pallas import tpu as pltpu
from jax.experimental.pallas import tpu_sc as plsc

_E = 64
_T = 32768
_ALPHA = 0.02
_NC, _NS, _L = 1, 16, 16
_NW = _NC * _NS
_IPW = _T // _NW
_EV = _E // _L

_BLKT = 8192
_GRID = _T // _BLKT
_LANES = 128

_mesh = plsc.VectorSubcoreMesh(core_axis_name="c", subcore_axis_name="s",
                               num_cores=_NC, num_subcores=_NS)


@functools.partial(
    pl.kernel,
    out_type=jax.ShapeDtypeStruct((_NW, _E), jnp.float32),
    mesh=_mesh,
    scratch_types=[
        pltpu.VMEM((_IPW,), jnp.int32),
        pltpu.VMEM((_L * _E,), jnp.float32),
        pltpu.VMEM((_E,), jnp.float32),
    ],
    compiler_params=pltpu.CompilerParams(needs_layout_passes=False),
)
def _hist(idx_hbm, counts_out, idx_v, hist_v, cnt_v):
    wid = lax.axis_index("s") * _NC + lax.axis_index("c")
    base = wid * _IPW
    pltpu.sync_copy(idx_hbm.at[pl.ds(base, _IPW)], idx_v)

    zero16 = jnp.zeros((_L,), jnp.float32)
    for i in range(_E):
        hist_v[pl.ds(i * _L, _L)] = zero16

    lane = lax.iota(jnp.int32, _L) * _E
    ones = jnp.ones((_L,), jnp.float32)
    for i in range(_IPW // _L):
        idx = idx_v[pl.ds(i * _L, _L)]
        plsc.addupdate_scatter(hist_v, [lane + idx], ones)

    acc = [zero16] * _EV
    for l in range(_L):
        for j in range(_EV):
            acc[j] = acc[j] + hist_v[pl.ds(l * _E + j * _L, _L)]
    for j in range(_EV):
        cnt_v[pl.ds(j * _L, _L)] = acc[j]

    pltpu.sync_copy(cnt_v, counts_out.at[wid])


def _colsum_body(probs_ref, out_ref):
    acc = jnp.zeros((_E, _LANES), jnp.float32)
    for j in range(_T // _LANES):
        acc = acc + probs_ref[:, j * _LANES:(j + 1) * _LANES]
    out_ref[...] = jnp.sum(acc, axis=1, keepdims=True).T


_colsum = pl.pallas_call(
    _colsum_body,
    in_specs=[pl.BlockSpec(memory_space=pltpu.VMEM)],
    out_specs=pl.BlockSpec(memory_space=pltpu.VMEM),
    out_shape=jax.ShapeDtypeStruct((1, _E), jnp.float32),
)


def kernel(router_probs, expert_idx):
    counts_part = _hist(expert_idx)
    colsum = _colsum(router_probs.T)
    weighted = counts_part * colsum
    s = jnp.sum(weighted)
    total = jnp.sum(counts_part)
    loss = (_ALPHA * _E / _T) * s / jnp.where(total < 1e-9, 1.0, total)
    return jnp.where(total < 1e-9, 0.0, loss)


# speedup vs baseline: 1.0221x; 1.0221x over previous
import functools

import jax
import jax.numpy as jnp
from jax import lax
from jax.experimental import pallas as pl
from jax.experimental.pallas import tpu as pltpu
from jax.experimental.pallas import tpu_sc as plsc

_E = 64
_T = 32768
_ALPHA = 0.02
_NC, _NS, _L = 1, 16, 16
_NW = _NC * _NS
_IPW = _T // _NW
_EV = _E // _L

_BLKT = 8192
_GRID = _T // _BLKT
_LANES = 128

_mesh = plsc.VectorSubcoreMesh(core_axis_name="c", subcore_axis_name="s",
                               num_cores=_NC, num_subcores=_NS)


@functools.partial(
    pl.kernel,
    out_type=jax.ShapeDtypeStruct((_NW, _E), jnp.float32),
    mesh=_mesh,
    scratch_types=[
        pltpu.VMEM((_IPW,), jnp.int32),
        pltpu.VMEM((_L * _E,), jnp.float32),
        pltpu.VMEM((_E,), jnp.float32),
    ],
    compiler_params=pltpu.CompilerParams(needs_layout_passes=False),
)
def _hist(idx_hbm, counts_out, idx_v, hist_v, cnt_v):
    wid = lax.axis_index("s") * _NC + lax.axis_index("c")
    base = wid * _IPW
    pltpu.sync_copy(idx_hbm.at[pl.ds(base, _IPW)], idx_v)

    zero16 = jnp.zeros((_L,), jnp.float32)

    def zbody(i, c):
        hist_v[pl.ds(i * _L, _L)] = zero16
        return c
    lax.fori_loop(0, _E, zbody, 0)

    lane = lax.iota(jnp.int32, _L) * _E
    ones = jnp.ones((_L,), jnp.float32)

    def hbody(i, c):
        for u in range(4):
            idx = idx_v[pl.ds(i * 4 * _L + u * _L, _L)]
            plsc.addupdate_scatter(hist_v, [lane + idx], ones)
        return c
    lax.fori_loop(0, _IPW // (4 * _L), hbody, 0)

    def cbody(l, acc):
        return tuple(acc[j] + hist_v[pl.ds(l * _E + j * _L, _L)]
                     for j in range(_EV))
    cnt = lax.fori_loop(0, _L, cbody, (zero16,) * _EV)
    for j in range(_EV):
        cnt_v[pl.ds(j * _L, _L)] = cnt[j]

    pltpu.sync_copy(cnt_v, counts_out.at[wid])


def _colsum_body(probs_ref, out_ref):
    acc = jnp.zeros((_E, _LANES), jnp.float32)
    for j in range(_T // _LANES):
        acc = acc + probs_ref[:, j * _LANES:(j + 1) * _LANES]
    out_ref[...] = jnp.sum(acc, axis=1, keepdims=True).T


_colsum = pl.pallas_call(
    _colsum_body,
    in_specs=[pl.BlockSpec(memory_space=pltpu.VMEM)],
    out_specs=pl.BlockSpec(memory_space=pltpu.VMEM),
    out_shape=jax.ShapeDtypeStruct((1, _E), jnp.float32),
)


def kernel(router_probs, expert_idx):
    counts_part = _hist(expert_idx)
    colsum = _colsum(router_probs.T)
    weighted = counts_part * colsum
    s = jnp.sum(weighted)
    total = jnp.sum(counts_part)
    loss = (_ALPHA * _E / _T) * s / jnp.where(total < 1e-9, 1.0, total)
    return jnp.where(total < 1e-9, 0.0, loss)
